# baseline (speedup 1.0000x reference)
import functools

import jax
import jax.numpy as jnp
from jax import lax
from jax.experimental import pallas as pl
from jax.experimental.pallas import tpu as pltpu
from jax.experimental.pallas import tpu_sc as plsc

B = 16384
EMBED = 16
NC = 2
NS = 16
NW = NC * NS
BPW = B // NW
CHUNK = 128
NCHUNK = BPW // CHUNK


def _sc_gather_partial(uemb, uidx, femb, fidx, ubias, fbias):
  mesh = plsc.VectorSubcoreMesh(core_axis_name="c", subcore_axis_name="s")

  @functools.partial(
      pl.kernel,
      mesh=mesh,
      compiler_params=pltpu.CompilerParams(use_tc_tiling_on_sc=False),
      out_type=(
          jax.ShapeDtypeStruct((NW, EMBED), jnp.float32),
          jax.ShapeDtypeStruct((B,), jnp.float32),
      ),
      scratch_types=[
          pltpu.VMEM((NCHUNK, CHUNK), jnp.int32),
          pltpu.VMEM((NCHUNK, CHUNK), jnp.int32),
          pltpu.VMEM((BPW, EMBED), jnp.float32),
          pltpu.VMEM((BPW, EMBED), jnp.float32),
          pltpu.VMEM((BPW,), jnp.float32),
          pltpu.VMEM((BPW,), jnp.float32),
          pltpu.VMEM((BPW,), jnp.float32),
          pltpu.VMEM((EMBED,), jnp.float32),
          pltpu.SemaphoreType.DMA,
      ],
  )
  def k(uemb_h, uidx_h, femb_h, fidx_h, ub_h, fb_h,
        part_h, bsum_h,
        uidx_v, fidx_v, urows_v, frows_v, ub_v, fb_v, bs_v, acc_v, sem):
    wid = lax.axis_index("s") * NC + lax.axis_index("c")
    base = wid * BPW
    pltpu.sync_copy(uidx_h.at[wid], uidx_v)
    pltpu.sync_copy(fidx_h.at[wid], fidx_v)

    copies = []
    for c in range(NCHUNK):
      sl = pl.ds(c * CHUNK, CHUNK)
      copies.append(pltpu.async_copy(uemb_h.at[uidx_v.at[c]], urows_v.at[sl], sem))
      copies.append(pltpu.async_copy(femb_h.at[fidx_v.at[c]], frows_v.at[sl], sem))
      copies.append(pltpu.async_copy(ub_h.at[uidx_v.at[c]], ub_v.at[sl], sem))
      copies.append(pltpu.async_copy(fb_h.at[fidx_v.at[c]], fb_v.at[sl], sem))
    for c in copies:
      c.wait()

    def body(i, accs):
      a0, a1, a2, a3 = accs
      r = i * 4
      a0 = a0 + urows_v[r, :] * frows_v[r, :]
      a1 = a1 + urows_v[r + 1, :] * frows_v[r + 1, :]
      a2 = a2 + urows_v[r + 2, :] * frows_v[r + 2, :]
      a3 = a3 + urows_v[r + 3, :] * frows_v[r + 3, :]
      return (a0, a1, a2, a3)

    z = jnp.zeros((EMBED,), jnp.float32)
    a0, a1, a2, a3 = lax.fori_loop(0, BPW // 4, body, (z, z, z, z))
    acc_v[:] = (a0 + a1) + (a2 + a3)

    for j in range(BPW // 16):
      sl = pl.ds(j * 16, 16)
      bs_v[sl] = ub_v[sl] + fb_v[sl]

    pltpu.sync_copy(acc_v, part_h.at[wid])
    pltpu.sync_copy(bs_v, bsum_h.at[pl.ds(base, BPW)])

  return k(uemb, uidx, femb, fidx, ubias, fbias)


def _tc_finish(partials, bsum):
  def body(p_ref, b_ref, o_ref):
    s = jnp.sum(p_ref[:])
    o_ref[:] = 1.0 / (1.0 + jnp.exp(-(b_ref[:] + s)))

  return pl.pallas_call(
      body,
      out_shape=jax.ShapeDtypeStruct((128, 128), jnp.float32),
  )(partials, bsum)


def kernel(inputs, user_embedding, user_bias, food_embedding, food_bias):
  uidx = inputs[:, 0].astype(jnp.int32).reshape(NW, NCHUNK, CHUNK)
  fidx = inputs[:, -1].astype(jnp.int32).reshape(NW, NCHUNK, CHUNK)
  part, bsum = _sc_gather_partial(
      user_embedding, uidx, food_embedding, fidx,
      user_bias.reshape(-1), food_bias.reshape(-1))
  out = _tc_finish(part.reshape(4, 128), bsum.reshape(128, 128))
  return out.reshape(B, 1)
